# baseline (speedup 1.0000x reference)
import jax
import jax.numpy as jnp
from jax.experimental import pallas as pl
from jax.experimental.pallas import tpu as pltpu

_ROWS, _D = 16384, 256
_BLK = 2048


def _select_body(mask_ref, base_ref, oov_ref, out_ref):
    m = mask_ref[...]
    out_ref[...] = jnp.where(m != 0, base_ref[...], oov_ref[...])


def kernel(base_embedding, in_vocab, oov_embedding):
    base_embedding = base_embedding.astype(jnp.float32)
    mask = in_vocab.astype(jnp.int32).reshape(_ROWS, 1)
    oov = oov_embedding.reshape(1, _D)
    grid = _ROWS // _BLK
    return pl.pallas_call(
        _select_body,
        grid=(grid,),
        in_specs=[
            pl.BlockSpec((_BLK, 1), lambda i: (i, 0)),
            pl.BlockSpec((_BLK, _D), lambda i: (i, 0)),
            pl.BlockSpec((1, _D), lambda i: (0, 0)),
        ],
        out_specs=pl.BlockSpec((_BLK, _D), lambda i: (i, 0)),
        out_shape=jax.ShapeDtypeStruct((_ROWS, _D), jnp.float32),
    )(mask, base_embedding, oov)
